# baseline (speedup 1.0000x reference)
import functools

import jax
import jax.numpy as jnp
from jax import lax
from jax.experimental import pallas as pl
from jax.experimental.pallas import tpu as pltpu
from jax.experimental.pallas import tpu_sc as plsc

B = 8
N = 511
N1 = 512
H = 8
V = 512
L = 16
NC, NS = 2, 16
NW = NC * NS
ROWS_PER_W = (B * N1) // NW
TILES_PER_B = N1 // ROWS_PER_W
G = 16
NGROUPS = ROWS_PER_W // G
GROUP_WORDS = G * N1


def _gather_row(src_buf, rowv, emb_v, iota, out_bufs, buf, lr):
    col0 = jnp.maximum(iota - 1, 0)
    v0 = plsc.load_gather(src_buf, [rowv, col0])
    v0 = jnp.where(iota == 0, 1, v0)
    out_bufs[buf, lr, pl.ds(0, L)] = plsc.load_gather(emb_v, [v0])
    for j in range(1, N1 // L):
        colj = iota + (j * L - 1)
        vals = plsc.load_gather(src_buf, [rowv, colj])
        out_bufs[buf, lr, pl.ds(j * L, L)] = plsc.load_gather(emb_v, [vals])


def _tile_body(pl_hbm, emb_hbm, out_hbm, idx_buf, out_bufs, emb_v,
               sem0, sem1, sem_stage):
    wid = lax.axis_index("s") * NC + lax.axis_index("c")
    b = wid // TILES_PER_B
    i_start = (wid % TILES_PER_B) * ROWS_PER_W
    start_p = jnp.maximum(i_start - 1, 0)

    pltpu.sync_copy(emb_hbm, emb_v)
    stage_head = 24
    pltpu.sync_copy(pl_hbm.at[pl.ds(start_p, stage_head), b, :],
                    idx_buf.at[pl.ds(0, stage_head), :])
    stage_rest = pltpu.async_copy(
        pl_hbm.at[pl.ds(start_p + stage_head, ROWS_PER_W - stage_head), b, :],
        idx_buf.at[pl.ds(stage_head, ROWS_PER_W - stage_head), :],
        sem_stage,
    )

    iota = lax.iota(jnp.int32, L)
    ones = jnp.full((L,), 1, jnp.int32)
    emb1 = plsc.load_gather(emb_v, [ones])

    @pl.when(i_start == 0)
    def _():
        for j in range(N1 // L):
            out_bufs[0, 0, pl.ds(j * L, L)] = emb1

    @pl.when(i_start > 0)
    def _():
        _gather_row(idx_buf, jnp.full((L,), 0, jnp.int32), emb_v, iota,
                    out_bufs, 0, 0)

    descs = [None] * NGROUPS
    for g in range(NGROUPS):
        buf = g & 1
        if g == 1:
            stage_rest.wait()
        if g >= 2:
            for d in descs[g - 2]:
                d.wait()
        lo = 1 if g == 0 else 0

        @pl.loop(lo, G, unroll=2)
        def _row(lr):
            i = i_start + g * G + lr
            rowv = jnp.full((L,), i - 1 - start_p, jnp.int32)
            _gather_row(idx_buf, rowv, emb_v, iota, out_bufs, buf, lr)

        row0 = i_start + g * G
        descs[g] = [
            pltpu.async_copy(
                out_bufs.at[buf],
                out_hbm.at[b * H + h, pl.ds(row0, G), :],
                sem0 if buf == 0 else sem1,
            )
            for h in range(H)
        ]
    for g in (NGROUPS - 2, NGROUPS - 1):
        for d in descs[g]:
            d.wait()


def kernel(X, mask, path_lengths, path_length_embedding):
    del X, mask
    emb_flat = path_length_embedding.reshape(-1)

    mesh = plsc.VectorSubcoreMesh(
        core_axis_name="c", subcore_axis_name="s", num_cores=NC, num_subcores=NS
    )
    run = functools.partial(
        pl.kernel,
        out_type=jax.ShapeDtypeStruct((B * H, N1, N1), jnp.float32),
        mesh=mesh,
        scratch_types=[
            pltpu.VMEM((ROWS_PER_W, N), jnp.int32),
            pltpu.VMEM((2, G, N1), jnp.float32),
            pltpu.VMEM((V,), jnp.float32),
            pltpu.SemaphoreType.DMA,
            pltpu.SemaphoreType.DMA,
            pltpu.SemaphoreType.DMA,
        ],
        compiler_params=pltpu.CompilerParams(needs_layout_passes=False),
    )(_tile_body)
    return run(jnp.transpose(path_lengths, (1, 0, 2)), emb_flat)


# speedup vs baseline: 1.3002x; 1.3002x over previous
import functools

import jax
import jax.numpy as jnp
from jax import lax
from jax.experimental import pallas as pl
from jax.experimental.pallas import tpu as pltpu
from jax.experimental.pallas import tpu_sc as plsc

B = 8
N = 511
N1 = 512
H = 8
V = 512
L = 16
NC, NS = 2, 16
NW = NC * NS
ROWS_PER_W = (B * N1) // NW
TILES_PER_B = N1 // ROWS_PER_W
G = 16
NGROUPS = ROWS_PER_W // G
GROUP_WORDS = G * N1


SUB = 8


def _gather_row(src_buf, rowv, emb_v, iota, out_bufs, buf, lr):
    for j0 in range(0, N1 // L, SUB):
        pvs = []
        for j in range(j0, j0 + SUB):
            if j == 0:
                col0 = jnp.maximum(iota - 1, 0)
                v0 = plsc.load_gather(src_buf, [rowv, col0])
                pvs.append(jnp.where(iota == 0, 1, v0))
            else:
                colj = iota + (j * L - 1)
                pvs.append(plsc.load_gather(src_buf, [rowv, colj]))
        for k, j in enumerate(range(j0, j0 + SUB)):
            out_bufs[buf, lr, pl.ds(j * L, L)] = plsc.load_gather(emb_v, [pvs[k]])


def _tile_body(pl_hbm, emb_hbm, out_hbm, idx_buf, out_bufs, emb_v,
               sem0, sem1, sem_stage):
    wid = lax.axis_index("s") * NC + lax.axis_index("c")
    b = wid // TILES_PER_B
    i_start = (wid % TILES_PER_B) * ROWS_PER_W
    start_p = jnp.maximum(i_start - 1, 0)

    pltpu.sync_copy(emb_hbm, emb_v)
    stage_head = 24
    pltpu.sync_copy(pl_hbm.at[pl.ds(start_p, stage_head), b, :],
                    idx_buf.at[pl.ds(0, stage_head), :])
    stage_rest = pltpu.async_copy(
        pl_hbm.at[pl.ds(start_p + stage_head, ROWS_PER_W - stage_head), b, :],
        idx_buf.at[pl.ds(stage_head, ROWS_PER_W - stage_head), :],
        sem_stage,
    )

    iota = lax.iota(jnp.int32, L)
    ones = jnp.full((L,), 1, jnp.int32)
    emb1 = plsc.load_gather(emb_v, [ones])

    @pl.when(i_start == 0)
    def _():
        for j in range(N1 // L):
            out_bufs[0, 0, pl.ds(j * L, L)] = emb1

    @pl.when(i_start > 0)
    def _():
        _gather_row(idx_buf, jnp.full((L,), 0, jnp.int32), emb_v, iota,
                    out_bufs, 0, 0)

    descs = [None] * NGROUPS
    for g in range(NGROUPS):
        buf = g & 1
        if g == 1:
            stage_rest.wait()
        if g >= 2:
            for d in descs[g - 2]:
                d.wait()
        lo = 1 if g == 0 else 0

        @pl.loop(lo, G)
        def _row(lr):
            i = i_start + g * G + lr
            rowv = jnp.full((L,), i - 1 - start_p, jnp.int32)
            _gather_row(idx_buf, rowv, emb_v, iota, out_bufs, buf, lr)

        row0 = i_start + g * G
        descs[g] = [
            pltpu.async_copy(
                out_bufs.at[buf],
                out_hbm.at[b * H + h, pl.ds(row0, G), :],
                sem0 if buf == 0 else sem1,
            )
            for h in range(H)
        ]
    for g in (NGROUPS - 2, NGROUPS - 1):
        for d in descs[g]:
            d.wait()


def kernel(X, mask, path_lengths, path_length_embedding):
    del X, mask
    emb_flat = path_length_embedding.reshape(-1)

    mesh = plsc.VectorSubcoreMesh(
        core_axis_name="c", subcore_axis_name="s", num_cores=NC, num_subcores=NS
    )
    run = functools.partial(
        pl.kernel,
        out_type=jax.ShapeDtypeStruct((B * H, N1, N1), jnp.float32),
        mesh=mesh,
        scratch_types=[
            pltpu.VMEM((ROWS_PER_W, N), jnp.int32),
            pltpu.VMEM((2, G, N1), jnp.float32),
            pltpu.VMEM((V,), jnp.float32),
            pltpu.SemaphoreType.DMA,
            pltpu.SemaphoreType.DMA,
            pltpu.SemaphoreType.DMA,
        ],
        compiler_params=pltpu.CompilerParams(needs_layout_passes=False),
    )(_tile_body)
    return run(jnp.transpose(path_lengths, (1, 0, 2)), emb_flat)


# speedup vs baseline: 1.3225x; 1.0171x over previous
import functools

import jax
import jax.numpy as jnp
from jax import lax
from jax.experimental import pallas as pl
from jax.experimental.pallas import tpu as pltpu
from jax.experimental.pallas import tpu_sc as plsc

B = 8
N = 511
N1 = 512
H = 8
V = 512
L = 16
NC, NS = 2, 16
NW = NC * NS
ROWS_PER_W = (B * N1) // NW
TILES_PER_B = N1 // ROWS_PER_W
G = 16
NGROUPS = ROWS_PER_W // G
GROUP_WORDS = G * N1


SUB = 16


def _gather_row(src_buf, rowv, emb_v, iota, out_bufs, buf, lr):
    for j0 in range(0, N1 // L, SUB):
        pvs = []
        for j in range(j0, j0 + SUB):
            if j == 0:
                col0 = jnp.maximum(iota - 1, 0)
                v0 = plsc.load_gather(src_buf, [rowv, col0])
                pvs.append(jnp.where(iota == 0, 1, v0))
            else:
                colj = iota + (j * L - 1)
                pvs.append(plsc.load_gather(src_buf, [rowv, colj]))
        for k, j in enumerate(range(j0, j0 + SUB)):
            out_bufs[buf, lr, pl.ds(j * L, L)] = plsc.load_gather(emb_v, [pvs[k]])


def _tile_body(pl_hbm, emb_hbm, out_hbm, idx_buf, out_bufs, emb_v,
               sem0, sem1, sem_stage):
    wid = lax.axis_index("s") * NC + lax.axis_index("c")
    b = wid // TILES_PER_B
    i_start = (wid % TILES_PER_B) * ROWS_PER_W
    start_p = jnp.maximum(i_start - 1, 0)

    pltpu.sync_copy(emb_hbm, emb_v)
    stage_head = 24
    pltpu.sync_copy(pl_hbm.at[pl.ds(start_p, stage_head), b, :],
                    idx_buf.at[pl.ds(0, stage_head), :])
    stage_rest = pltpu.async_copy(
        pl_hbm.at[pl.ds(start_p + stage_head, ROWS_PER_W - stage_head), b, :],
        idx_buf.at[pl.ds(stage_head, ROWS_PER_W - stage_head), :],
        sem_stage,
    )

    iota = lax.iota(jnp.int32, L)
    ones = jnp.full((L,), 1, jnp.int32)
    emb1 = plsc.load_gather(emb_v, [ones])

    @pl.when(i_start == 0)
    def _():
        for j in range(N1 // L):
            out_bufs[0, 0, pl.ds(j * L, L)] = emb1

    @pl.when(i_start > 0)
    def _():
        _gather_row(idx_buf, jnp.full((L,), 0, jnp.int32), emb_v, iota,
                    out_bufs, 0, 0)

    descs = [None] * NGROUPS
    for g in range(NGROUPS):
        buf = g & 1
        if g == 1:
            stage_rest.wait()
        if g >= 2:
            for d in descs[g - 2]:
                d.wait()
        lo = 1 if g == 0 else 0

        @pl.loop(lo, G)
        def _row(lr):
            i = i_start + g * G + lr
            rowv = jnp.full((L,), i - 1 - start_p, jnp.int32)
            _gather_row(idx_buf, rowv, emb_v, iota, out_bufs, buf, lr)

        row0 = i_start + g * G
        descs[g] = [
            pltpu.async_copy(
                out_bufs.at[buf],
                out_hbm.at[b * H + h, pl.ds(row0, G), :],
                sem0 if buf == 0 else sem1,
            )
            for h in range(H)
        ]
    for g in (NGROUPS - 2, NGROUPS - 1):
        for d in descs[g]:
            d.wait()


def kernel(X, mask, path_lengths, path_length_embedding):
    del X, mask
    emb_flat = path_length_embedding.reshape(-1)

    mesh = plsc.VectorSubcoreMesh(
        core_axis_name="c", subcore_axis_name="s", num_cores=NC, num_subcores=NS
    )
    run = functools.partial(
        pl.kernel,
        out_type=jax.ShapeDtypeStruct((B * H, N1, N1), jnp.float32),
        mesh=mesh,
        scratch_types=[
            pltpu.VMEM((ROWS_PER_W, N), jnp.int32),
            pltpu.VMEM((2, G, N1), jnp.float32),
            pltpu.VMEM((V,), jnp.float32),
            pltpu.SemaphoreType.DMA,
            pltpu.SemaphoreType.DMA,
            pltpu.SemaphoreType.DMA,
        ],
        compiler_params=pltpu.CompilerParams(needs_layout_passes=False),
    )(_tile_body)
    return run(jnp.transpose(path_lengths, (1, 0, 2)), emb_flat)


# speedup vs baseline: 1.3491x; 1.0201x over previous
import functools

import jax
import jax.numpy as jnp
from jax import lax
from jax.experimental import pallas as pl
from jax.experimental.pallas import tpu as pltpu
from jax.experimental.pallas import tpu_sc as plsc

B = 8
N = 511
N1 = 512
H = 8
V = 512
L = 16
NC, NS = 2, 16
NW = NC * NS
ROWS_PER_W = (B * N1) // NW
TILES_PER_B = N1 // ROWS_PER_W
G = 32
NGROUPS = ROWS_PER_W // G
GROUP_WORDS = G * N1


SUB = 8


def _gather_row(src_buf, rowv, emb_v, iota, out_bufs, buf, lr):
    for j0 in range(0, N1 // L, SUB):
        pvs = []
        for j in range(j0, j0 + SUB):
            if j == 0:
                col0 = jnp.maximum(iota - 1, 0)
                v0 = plsc.load_gather(src_buf, [rowv, col0])
                pvs.append(jnp.where(iota == 0, 1, v0))
            else:
                colj = iota + (j * L - 1)
                pvs.append(plsc.load_gather(src_buf, [rowv, colj]))
        for k, j in enumerate(range(j0, j0 + SUB)):
            out_bufs[buf, lr, pl.ds(j * L, L)] = plsc.load_gather(emb_v, [pvs[k]])


def _tile_body(pl_hbm, emb_hbm, out_hbm, idx_buf, out_bufs, emb_v,
               sem0, sem1, sem_stage):
    wid = lax.axis_index("s") * NC + lax.axis_index("c")
    b = wid // TILES_PER_B
    i_start = (wid % TILES_PER_B) * ROWS_PER_W
    start_p = jnp.maximum(i_start - 1, 0)

    pltpu.sync_copy(emb_hbm, emb_v)
    stage_head = 24
    pltpu.sync_copy(pl_hbm.at[pl.ds(start_p, stage_head), b, :],
                    idx_buf.at[pl.ds(0, stage_head), :])
    stage_rest = pltpu.async_copy(
        pl_hbm.at[pl.ds(start_p + stage_head, ROWS_PER_W - stage_head), b, :],
        idx_buf.at[pl.ds(stage_head, ROWS_PER_W - stage_head), :],
        sem_stage,
    )

    iota = lax.iota(jnp.int32, L)
    ones = jnp.full((L,), 1, jnp.int32)
    emb1 = plsc.load_gather(emb_v, [ones])

    @pl.when(i_start == 0)
    def _():
        for j in range(N1 // L):
            out_bufs[0, 0, pl.ds(j * L, L)] = emb1

    descs = [None] * NGROUPS
    for g in range(NGROUPS):
        buf = g & 1
        if g == 1:
            stage_rest.wait()
        if g >= 2:
            for d in descs[g - 2]:
                d.wait()
        lo = jnp.where(i_start == 0, 1, 0) if g == 0 else 0

        @pl.loop(lo, G)
        def _row(lr):
            i = i_start + g * G + lr
            rowv = jnp.full((L,), i - 1 - start_p, jnp.int32)
            _gather_row(idx_buf, rowv, emb_v, iota, out_bufs, buf, lr)

        row0 = i_start + g * G
        descs[g] = [
            pltpu.async_copy(
                out_bufs.at[buf],
                out_hbm.at[b * H + h, pl.ds(row0, G), :],
                sem0 if buf == 0 else sem1,
            )
            for h in range(H)
        ]
    for g in (NGROUPS - 2, NGROUPS - 1):
        for d in descs[g]:
            d.wait()


def kernel(X, mask, path_lengths, path_length_embedding):
    del X, mask
    emb_flat = path_length_embedding.reshape(-1)

    mesh = plsc.VectorSubcoreMesh(
        core_axis_name="c", subcore_axis_name="s", num_cores=NC, num_subcores=NS
    )
    run = functools.partial(
        pl.kernel,
        out_type=jax.ShapeDtypeStruct((B * H, N1, N1), jnp.float32),
        mesh=mesh,
        scratch_types=[
            pltpu.VMEM((ROWS_PER_W, N), jnp.int32),
            pltpu.VMEM((2, G, N1), jnp.float32),
            pltpu.VMEM((V,), jnp.float32),
            pltpu.SemaphoreType.DMA,
            pltpu.SemaphoreType.DMA,
            pltpu.SemaphoreType.DMA,
        ],
        compiler_params=pltpu.CompilerParams(needs_layout_passes=False),
    )(_tile_body)
    return run(jnp.transpose(path_lengths, (1, 0, 2)), emb_flat)
